# Initial kernel scaffold; baseline (speedup 1.0000x reference)
import jax
import jax.numpy as jnp
from jax.experimental import pallas as pl


def kernel(board_tensor, metadata, piece_table, square_table, turn_table, castling_table, en_passant_table, rms_weight):
    raise NotImplementedError("write your pallas kernel here")


# speedup vs baseline: 1.8340x; 1.8340x over previous
import functools

import jax
import jax.numpy as jnp
from jax import lax
from jax.experimental import pallas as pl
from jax.experimental.pallas import tpu as pltpu
from jax.experimental.pallas import tpu_sc as plsc

EMBED_DIM = 128
EPS = 1e-06

TURN_OFF = 832
CASTLE_OFF = TURN_OFF + 2
EP_OFF = CASTLE_OFF + 16
ZERO_ROW = EP_OFF + 65
TABLE_ROWS = 928


def _table_body(piece_ref, square_ref, turn_ref, castle_ref, ep_ref, w_ref, out_ref):
    piece = piece_ref[...]
    square = square_ref[...]
    comb = (square[:, None, :] + piece[None, :, :]).reshape(832, EMBED_DIM)
    zeros = jnp.zeros((TABLE_ROWS - ZERO_ROW, EMBED_DIM), jnp.float32)
    rows = jnp.concatenate(
        [comb, turn_ref[...], castle_ref[...], ep_ref[...], zeros], axis=0)
    ms = jnp.mean(rows * rows, axis=1, keepdims=True)
    out_ref[...] = rows * lax.rsqrt(ms + EPS) * w_ref[...]


def _prep_table(piece, square, turn, castle, ep, w):
    return pl.pallas_call(
        _table_body,
        out_shape=jax.ShapeDtypeStruct((TABLE_ROWS, EMBED_DIM), jnp.float32),
    )(piece, square, turn, castle, ep, w.reshape(1, EMBED_DIM))


def _idx_body(board_ref, meta_ref, out_ref):
    board = board_ref[...]
    offs = lax.broadcasted_iota(jnp.int32, (1, 64), 1) * 13
    m = meta_ref[...]
    cls = jnp.full((board.shape[0], 1), ZERO_ROW, jnp.int32)
    out_ref[...] = jnp.concatenate(
        [cls, board + offs,
         m[:, 0:1] + TURN_OFF, m[:, 1:2] + CASTLE_OFF, m[:, 2:3] + EP_OFF],
        axis=1)


def _prep_idx(board, meta):
    b = board.shape[0]
    blk = 2048
    assert b % blk == 0
    return pl.pallas_call(
        _idx_body,
        grid=(b // blk,),
        in_specs=[pl.BlockSpec((blk, 64), lambda i: (i, 0)),
                  pl.BlockSpec((blk, 3), lambda i: (i, 0))],
        out_specs=pl.BlockSpec((blk, 68), lambda i: (i, 0)),
        out_shape=jax.ShapeDtypeStruct((b, 68), jnp.int32),
    )(board, meta)


def _sc_gather(table, idx2d, total_rows):
    info = plsc.get_sparse_core_info()
    nc, ns = info.num_cores, info.num_subcores
    nw = nc * ns
    k = 128
    chunks_total = idx2d.shape[0]
    assert chunks_total % nw == 0
    chunks = chunks_total // nw
    per_w = chunks * k

    mesh = plsc.VectorSubcoreMesh(core_axis_name="c", subcore_axis_name="s")

    @functools.partial(
        pl.kernel,
        out_type=jax.ShapeDtypeStruct((total_rows, EMBED_DIM), jnp.float32),
        mesh=mesh,
        scratch_types=[
            pltpu.VMEM((chunks, k), jnp.int32),
            pltpu.VMEM((k, EMBED_DIM), jnp.float32),
            pltpu.SemaphoreType.DMA,
        ],
    )
    def gather_kernel(table_hbm, idx_hbm, out_hbm, idx_v, rows_v, sem):
        wid = lax.axis_index("s") * nc + lax.axis_index("c")
        pltpu.sync_copy(idx_hbm.at[pl.ds(wid * chunks, chunks)], idx_v)
        base = wid * per_w

        def step(j, carry):
            pltpu.async_copy(table_hbm.at[idx_v.at[j]], rows_v, sem).wait()
            pltpu.sync_copy(rows_v, out_hbm.at[pl.ds(base + j * k, k)])
            return carry

        lax.fori_loop(0, chunks, step, 0)

    return gather_kernel(table, idx2d)


def kernel(board_tensor, metadata, piece_table, square_table, turn_table,
           castling_table, en_passant_table, rms_weight):
    b = board_tensor.shape[0]
    board = board_tensor.astype(jnp.int32)
    meta = metadata.astype(jnp.int32)

    table = _prep_table(piece_table, square_table, turn_table,
                        castling_table, en_passant_table, rms_weight)
    idx = _prep_idx(board, meta)
    total_rows = b * 68
    idx2d = idx.reshape(total_rows // 128, 128)
    out = _sc_gather(table, idx2d, total_rows)
    return out.reshape(b, 68, EMBED_DIM)
